# baseline (speedup 1.0000x reference)
import functools

import jax
import jax.numpy as jnp
from jax.experimental import pallas as pl
from jax.experimental.pallas import tpu as pltpu


def _pool_body(bb, c, x_ref, ones_ref, w_ref, o_ref):
    x = x_ref[...]
    ss = jax.lax.dot(x, ones_ref[...],
                     precision=jax.lax.Precision.HIGHEST,
                     preferred_element_type=jnp.float32)
    m = ss.shape[-1]
    prod = ss.reshape(bb, c, m) * w_ref[...][None, :, :]
    i = pl.program_id(0)
    o_ref[pl.ds(i * bb, bb), :] = jnp.sum(prod, axis=1)


def _routing_body(l_ref, coef_ref, g_ref, loss_ref):
    logits = l_ref[...]
    B, M = logits.shape
    col = jax.lax.broadcasted_iota(jnp.int32, (B, M), 1)
    big = jnp.int32(M)

    m1 = jnp.max(logits, axis=1, keepdims=True)
    idx1 = jnp.min(jnp.where(logits == m1, col, big), axis=1, keepdims=True)
    masked = jnp.where(col == idx1, -jnp.inf, logits)
    m2 = jnp.max(masked, axis=1, keepdims=True)
    idx2 = jnp.min(jnp.where(masked == m2, col, big), axis=1, keepdims=True)

    e = jnp.exp(m2 - m1)
    denom = 1.0 + e
    g1 = 1.0 / denom
    g2 = e / denom
    gates = (jnp.where(col == idx1, g1, 0.0)
             + jnp.where(col == idx2, g2, 0.0))
    g_ref[...] = gates

    imp = jnp.sum(gates, axis=0, keepdims=True)
    load = jnp.sum((gates > 0.0).astype(jnp.float32), axis=0,
                   keepdims=True)

    def cv_sq(x):
        mean = jnp.sum(x) * jnp.float32(1.0 / M)
        var = jnp.sum((x - mean) ** 2) * jnp.float32(1.0 / (M - 1))
        return var / (mean * mean + jnp.float32(1e-10))

    loss_ref[0, 0] = (cv_sq(imp) + cv_sq(load)) * coef_ref[0]


def kernel(feats, w_gate, w_noise, loss_coef=0.01, noise_epsilon=0.01):
    B, C, H, W = feats.shape
    S = H * W
    M = w_gate.shape[1]
    x = feats.reshape(B * C, S)
    BB = 4
    ones_mat = jnp.full((S, M), 1.0 / S, dtype=jnp.float32)

    logits = pl.pallas_call(
        functools.partial(_pool_body, BB, C),
        grid=(B // BB,),
        in_specs=[
            pl.BlockSpec((BB * C, S), lambda i: (i, 0)),
            pl.BlockSpec((S, M), lambda i: (0, 0)),
            pl.BlockSpec((C, M), lambda i: (0, 0)),
        ],
        out_specs=pl.BlockSpec((B, M), lambda i: (0, 0)),
        out_shape=jax.ShapeDtypeStruct((B, M), jnp.float32),
    )(x, ones_mat, w_gate)

    coef = jnp.reshape(jnp.asarray(loss_coef, jnp.float32), (1,))
    gates, loss = pl.pallas_call(
        _routing_body,
        in_specs=[
            pl.BlockSpec(memory_space=pltpu.VMEM),
            pl.BlockSpec(memory_space=pltpu.SMEM),
        ],
        out_specs=[
            pl.BlockSpec(memory_space=pltpu.VMEM),
            pl.BlockSpec(memory_space=pltpu.SMEM),
        ],
        out_shape=[
            jax.ShapeDtypeStruct((B, M), jnp.float32),
            jax.ShapeDtypeStruct((1, 1), jnp.float32),
        ],
    )(logits, coef)

    return gates, loss[0, 0]


# speedup vs baseline: 1.1925x; 1.1925x over previous
import functools

import jax
import jax.numpy as jnp
from jax.experimental import pallas as pl
from jax.experimental.pallas import tpu as pltpu


def _pool_body(bb, c, x_ref, wt_ref, o_ref):
    x = x_ref[...]
    rows = bb * c
    m = wt_ref.shape[0]
    p = x[:, 0:128] + x[:, 128:256] + x[:, 256:384] + x[:, 384:512]
    tail = x[:, 448:576]
    lane = jax.lax.broadcasted_iota(jnp.int32, (rows, 128), 1)
    q = p + jnp.where(lane >= 64, tail, 0.0)
    t = q.T
    s = jnp.sum(t, axis=0)
    prod = wt_ref[...] * s[None, :]
    cols = []
    for b in range(bb):
        seg = prod[:, b * c:(b + 1) * c]
        f = seg[:, 0:128]
        for j in range(1, c // 128):
            f = f + seg[:, j * 128:(j + 1) * 128]
        width = 128
        while width > 1:
            h = width // 2
            f = f[:, 0:h] + f[:, h:width]
            width = h
        cols.append(f)
    blk = jnp.concatenate(cols, axis=1)
    i = pl.program_id(0)
    o_ref[pl.ds(i * bb, bb), :] = blk.T


def _routing_body(l_ref, coef_ref, g_ref, loss_ref):
    logits = l_ref[...]
    b_dim, m_dim = logits.shape
    col = jax.lax.broadcasted_iota(jnp.int32, (b_dim, m_dim), 1)
    big = jnp.int32(m_dim)

    m1 = jnp.max(logits, axis=1, keepdims=True)
    idx1 = jnp.min(jnp.where(logits == m1, col, big), axis=1, keepdims=True)
    masked = jnp.where(col == idx1, -jnp.inf, logits)
    m2 = jnp.max(masked, axis=1, keepdims=True)
    idx2 = jnp.min(jnp.where(masked == m2, col, big), axis=1, keepdims=True)

    e = jnp.exp(m2 - m1)
    denom = 1.0 + e
    g1 = 1.0 / denom
    g2 = e / denom
    gates = (jnp.where(col == idx1, g1, 0.0)
             + jnp.where(col == idx2, g2, 0.0))
    g_ref[...] = gates

    imp = jnp.sum(gates, axis=0, keepdims=True)
    load = jnp.sum((gates > 0.0).astype(jnp.float32), axis=0,
                   keepdims=True)

    def cv_sq(v):
        mean = jnp.sum(v) * jnp.float32(1.0 / m_dim)
        var = jnp.sum((v - mean) ** 2) * jnp.float32(1.0 / (m_dim - 1))
        return var / (mean * mean + jnp.float32(1e-10))

    loss_ref[0, 0] = (cv_sq(imp) + cv_sq(load)) * coef_ref[0]


def kernel(feats, w_gate, w_noise, loss_coef=0.01, noise_epsilon=0.01):
    B, C, H, W = feats.shape
    S = H * W
    M = w_gate.shape[1]
    x = feats.reshape(B * C, S)
    BB = 4
    wt = jnp.tile(w_gate.T * jnp.float32(1.0 / S), (1, BB))

    logits = pl.pallas_call(
        functools.partial(_pool_body, BB, C),
        grid=(B // BB,),
        in_specs=[
            pl.BlockSpec((BB * C, S), lambda i: (i, 0)),
            pl.BlockSpec((M, BB * C), lambda i: (0, 0)),
        ],
        out_specs=pl.BlockSpec((B, M), lambda i: (0, 0)),
        out_shape=jax.ShapeDtypeStruct((B, M), jnp.float32),
    )(x, wt)

    coef = jnp.reshape(jnp.asarray(loss_coef, jnp.float32), (1,))
    gates, loss = pl.pallas_call(
        _routing_body,
        in_specs=[
            pl.BlockSpec(memory_space=pltpu.VMEM),
            pl.BlockSpec(memory_space=pltpu.SMEM),
        ],
        out_specs=[
            pl.BlockSpec(memory_space=pltpu.VMEM),
            pl.BlockSpec(memory_space=pltpu.SMEM),
        ],
        out_shape=[
            jax.ShapeDtypeStruct((B, M), jnp.float32),
            jax.ShapeDtypeStruct((1, 1), jnp.float32),
        ],
    )(logits, coef)

    return gates, loss[0, 0]
